# baseline (speedup 1.0000x reference)
import jax
import jax.numpy as jnp
from jax import lax
from jax.experimental import pallas as pl
from jax.experimental.pallas import tpu as pltpu


def _make_row_kernel(T, Tp, T1, T2, W1, W2, OFF):
    def row_kernel(pk_ref, t1_ref, tw2_ref, w1_ref, b1_ref, b2_ref, out_ref):
        A = pk_ref[0]
        Np = Tp // 8
        Ar = A.reshape(Np, 8, 5)
        d8 = Ar[:, :, 1]
        v8 = Ar[:, :, 0]
        gsub = lax.broadcasted_iota(jnp.int32, (Np, 8), 0)
        glane = lax.broadcasted_iota(jnp.int32, (Np, 8), 1)
        g = gsub * 8 + glane
        maxd = jnp.max(d8)
        idx = jnp.min(jnp.where(d8 == maxd, g, Tp))
        cnt = jnp.sum(jnp.where((g >= idx) & (v8 != 0), 1, 0))

        g1_3 = lax.broadcasted_iota(jnp.int32, (W1, 8, 5), 0) * 8 + \
            lax.broadcasted_iota(jnp.int32, (W1, 8, 5), 1)
        A1 = jnp.where(g1_3 < idx, Ar[:W1], 0)

        A2f = pk_ref[0, pl.ds(idx, T2), :]
        g2_3 = lax.broadcasted_iota(jnp.int32, (W2, 8, 5), 0) * 8 + \
            lax.broadcasted_iota(jnp.int32, (W2, 8, 5), 1)
        A2 = jnp.where(g2_3 < cnt, A2f.reshape(W2, 8, 5), 0)

        iota_r = lax.broadcasted_iota(jnp.int32, (1, 256), 1)

        def onehot(Axs, Tn):
            O = jnp.zeros((Tn, 256), jnp.float32)
            for k in range(5):
                ik = Axs[:, k:k + 1] + OFF[k]
                O = O + jnp.where(ik == iota_r, 1.0, 0.0)
            return O

        acc2 = jnp.zeros((W2, 32), jnp.float32) + b2_ref[:, :]
        for s in range(8):
            O2s = onehot(A2[:, s, :], W2)
            acc2 = acc2 + jnp.dot(O2s, tw2_ref[s * 256:(s + 1) * 256, :],
                                  preferred_element_type=jnp.float32)

        v1_8 = A1[:, :, 0]
        m8 = jnp.where(v1_8 == 2, 1.0, 0.0)
        iu_r = lax.broadcasted_iota(jnp.int32, (W1, 1), 0)
        iu_c = lax.broadcasted_iota(jnp.int32, (1, W1), 1)
        Lstrict = jnp.where(iu_c < iu_r, 1.0, 0.0)
        ones8 = jnp.ones((8, 1), jnp.float32)
        rowtot = jnp.dot(m8, ones8, preferred_element_type=jnp.float32)
        prefix = jnp.dot(Lstrict, rowtot, preferred_element_type=jnp.float32)
        is_r = lax.broadcasted_iota(jnp.int32, (8, 1), 0)
        is_c = lax.broadcasted_iota(jnp.int32, (1, 8), 1)
        U8 = jnp.where(is_r < is_c, 1.0, 0.0)
        within = jnp.dot(m8, U8, preferred_element_type=jnp.float32)
        pci = (prefix + within).astype(jnp.int32)

        iota_w2 = lax.broadcasted_iota(jnp.int32, (1, W2), 1)
        acc1 = jnp.zeros((W1, 256), jnp.float32) + b1_ref[:, :]
        for s in range(8):
            E1s = jnp.dot(onehot(A1[:, s, :], W1), t1_ref[:, :],
                          preferred_element_type=jnp.float32)
            OHs = jnp.where(pci[:, s:s + 1] == iota_w2, 1.0, 0.0)
            subs = jnp.dot(OHs, acc2, preferred_element_type=jnp.float32)
            masks = v1_8[:, s:s + 1] == 2
            xs = jnp.where(masks, subs, E1s)
            acc1 = acc1 + jnp.dot(xs, w1_ref[s * 32:(s + 1) * 32, :],
                                  preferred_element_type=jnp.float32)
        out_ref[0] = acc1

    return row_kernel


def kernel(value, depth, pos, ve1, de1, se1, ve2, de2, se2, conv1_w, conv1_b, conv2_w, conv2_b):
    B, T = value.shape
    T1 = 512 + 32 * (B - 1)
    T2 = 4 * T1
    W1 = T1 // 8
    W2 = T2 // 8
    Tp = T + T2

    v32 = value.astype(jnp.int32)
    d32 = depth.astype(jnp.int32)
    p32 = pos.astype(jnp.int32)
    packed = jnp.concatenate([v32[:, :, None], d32[:, :, None], p32], axis=2)
    packed = jnp.pad(packed, ((0, 0), (0, T2), (0, 0)))

    nv = ve1.shape[0]
    nd = de1.shape[0]
    ns = se1.shape[1]
    OFF = (0, nv, nv + nd, nv + nd + ns, nv + nd + 2 * ns)

    def pack_tab(ve, de, se):
        t = jnp.concatenate([ve, de, se[0], se[1], se[2]], axis=0)
        return jnp.pad(t, ((0, 256 - t.shape[0]), (0, 0)))

    tab1 = pack_tab(ve1, de1, se1)
    tab2 = pack_tab(ve2, de2, se2)
    w1r = jnp.transpose(conv1_w, (2, 1, 0)).reshape(8 * conv1_w.shape[1], conv1_w.shape[0])
    w2r = jnp.transpose(conv2_w, (2, 1, 0)).reshape(8 * conv2_w.shape[1], conv2_w.shape[0])
    tabw2 = jnp.concatenate(
        [jnp.dot(tab2, w2r[s * 32:(s + 1) * 32, :]) for s in range(8)], axis=0)
    b1 = conv1_b.reshape(1, -1)
    b2 = conv2_b.reshape(1, -1)

    row_kernel = _make_row_kernel(T, Tp, T1, T2, W1, W2, OFF)
    out = pl.pallas_call(
        row_kernel,
        grid=(B,),
        in_specs=[
            pl.BlockSpec((1, Tp, 5), lambda i: (i, 0, 0)),
            pl.BlockSpec((256, 32), lambda i: (0, 0)),
            pl.BlockSpec((8 * 256, 32), lambda i: (0, 0)),
            pl.BlockSpec((256, 256), lambda i: (0, 0)),
            pl.BlockSpec((1, 256), lambda i: (0, 0)),
            pl.BlockSpec((1, 32), lambda i: (0, 0)),
        ],
        out_specs=pl.BlockSpec((1, W1, 256), lambda i: (i, 0, 0)),
        out_shape=jax.ShapeDtypeStruct((B, W1, 256), jnp.float32),
        compiler_params=pltpu.CompilerParams(
            dimension_semantics=("parallel",)),
    )(packed, tab1, tabw2, w1r, b1, b2)
    return out


# speedup vs baseline: 1.8694x; 1.8694x over previous
import jax
import jax.numpy as jnp
from jax import lax
from jax.experimental import pallas as pl
from jax.experimental.pallas import tpu as pltpu


def _make_row_kernel(T, T1, T2, W1, W2, ND, OFF):
    NB = max(1, (T - 1).bit_length())

    def row_kernel(pk_ref, t1_ref, t2_ref, w2c_ref, w1_ref, b1_ref, b2_ref, out_ref):
        A = pk_ref[0]
        iota_T = lax.broadcasted_iota(jnp.int32, (T, 1), 0)
        d = A[:, 1:2]
        maxd = jnp.max(d)
        idx = jnp.min(jnp.where(d == maxd, iota_T, T))
        v = A[:, 0:1]
        cnt = jnp.sum(jnp.where((iota_T >= idx) & (v != 0), 1, 0))

        iota1 = lax.broadcasted_iota(jnp.int32, (T1, 1), 0)
        A1 = jnp.where(iota1 < idx, A[:T1, :], 0)

        Ash = A
        for k in range(NB):
            sh = jnp.roll(Ash, -(1 << k), axis=0)
            bit = (idx >> k) & 1
            Ash = jnp.where(bit == 1, sh, Ash)
        iota2 = lax.broadcasted_iota(jnp.int32, (T2, 1), 0)
        A2 = jnp.where(iota2 < cnt, Ash[:T2, :], 0)

        iota_r = lax.broadcasted_iota(jnp.int32, (1, 256), 1)

        def embed(Ax, Tn, tab):
            ivd = Ax[:, 0:1] * ND + Ax[:, 1:2]
            O = jnp.where(ivd == iota_r, 1.0, 0.0)
            for k in range(3):
                ik = Ax[:, 2 + k:3 + k] + OFF[k]
                O = O + jnp.where(ik == iota_r, 1.0, 0.0)
            return jnp.dot(O, tab, preferred_element_type=jnp.float32)

        x = embed(A1, T1, t1_ref[:, :])
        y = embed(A2, T2, t2_ref[:, :])

        Z = jnp.dot(y, w2c_ref[:, :], preferred_element_type=jnp.float32)
        tmod = iota2 - (iota2 // 8) * 8
        Zsel = jnp.zeros((T2, 32), jnp.float32)
        for s in range(8):
            Zsel = Zsel + jnp.where(tmod == s, 1.0, 0.0) * Z[:, s * 32:(s + 1) * 32]
        iota_w2 = lax.broadcasted_iota(jnp.int32, (W2, 1), 0)
        iota_t2l = lax.broadcasted_iota(jnp.int32, (1, T2), 1)
        G = jnp.where(iota_t2l // 8 == iota_w2, 1.0, 0.0)
        acc2 = jnp.dot(G, Zsel, preferred_element_type=jnp.float32) + b2_ref[:, :]

        mask2 = A1[:, 0:1] == 2
        mf = jnp.where(mask2, 1.0, 0.0)
        iota_c1 = lax.broadcasted_iota(jnp.int32, (1, T1), 1)
        Ltri = jnp.where(iota_c1 < iota1, 1.0, 0.0)
        pcum = jnp.dot(Ltri, mf, preferred_element_type=jnp.float32)
        pci = pcum.astype(jnp.int32)
        iota_w2r = lax.broadcasted_iota(jnp.int32, (1, W2), 1)
        OH = jnp.where(pci == iota_w2r, 1.0, 0.0)
        sub = jnp.dot(OH, acc2, preferred_element_type=jnp.float32)
        x = jnp.where(mask2, sub, x)

        iota_w1 = lax.broadcasted_iota(jnp.int32, (W1, 1), 0)
        iota_t1 = lax.broadcasted_iota(jnp.int32, (1, T1), 1)
        acc1 = jnp.zeros((W1, 256), jnp.float32) + b1_ref[:, :]
        for s in range(8):
            P = jnp.where(iota_t1 == iota_w1 * 8 + s, 1.0, 0.0)
            xs = jnp.dot(P, x, preferred_element_type=jnp.float32)
            acc1 = acc1 + jnp.dot(xs, w1_ref[s * 32:(s + 1) * 32, :],
                                  preferred_element_type=jnp.float32)
        out_ref[0] = acc1

    return row_kernel


def kernel(value, depth, pos, ve1, de1, se1, ve2, de2, se2, conv1_w, conv1_b, conv2_w, conv2_b):
    B, T = value.shape
    T1 = 512 + 32 * (B - 1)
    T2 = 4 * T1
    W1 = T1 // 8
    W2 = T2 // 8

    v32 = value.astype(jnp.int32)
    d32 = depth.astype(jnp.int32)
    p32 = pos.astype(jnp.int32)
    packed = jnp.concatenate([v32[:, :, None], d32[:, :, None], p32], axis=2)

    nv = ve1.shape[0]
    nd = de1.shape[0]
    ns = se1.shape[1]
    nvd = nv * nd
    OFF = (nvd, nvd + ns, nvd + 2 * ns)

    def pack_tab(ve, de, se):
        vd = (ve[:, None, :] + de[None, :, :]).reshape(nvd, ve.shape[1])
        t = jnp.concatenate([vd, se[0], se[1], se[2]], axis=0)
        return jnp.pad(t, ((0, 256 - t.shape[0]), (0, 0)))

    tab1 = pack_tab(ve1, de1, se1)
    tab2 = pack_tab(ve2, de2, se2)
    w1r = jnp.transpose(conv1_w, (2, 1, 0)).reshape(8 * conv1_w.shape[1], conv1_w.shape[0])
    w2c = jnp.transpose(conv2_w, (1, 2, 0)).reshape(conv2_w.shape[1], -1)
    b1 = conv1_b.reshape(1, -1)
    b2 = conv2_b.reshape(1, -1)

    row_kernel = _make_row_kernel(T, T1, T2, W1, W2, nd, OFF)
    out = pl.pallas_call(
        row_kernel,
        grid=(B,),
        in_specs=[
            pl.BlockSpec((1, T, 5), lambda i: (i, 0, 0)),
            pl.BlockSpec((256, 32), lambda i: (0, 0)),
            pl.BlockSpec((256, 32), lambda i: (0, 0)),
            pl.BlockSpec((32, 256), lambda i: (0, 0)),
            pl.BlockSpec((256, 256), lambda i: (0, 0)),
            pl.BlockSpec((1, 256), lambda i: (0, 0)),
            pl.BlockSpec((1, 32), lambda i: (0, 0)),
        ],
        out_specs=pl.BlockSpec((1, W1, 256), lambda i: (i, 0, 0)),
        out_shape=jax.ShapeDtypeStruct((B, W1, 256), jnp.float32),
        compiler_params=pltpu.CompilerParams(
            dimension_semantics=("parallel",)),
    )(packed, tab1, tab2, w2c, w1r, b1, b2)
    return out


# speedup vs baseline: 2.0521x; 1.0978x over previous
import jax
import jax.numpy as jnp
from jax import lax
from jax.experimental import pallas as pl
from jax.experimental.pallas import tpu as pltpu


def _make_row_kernel(T, T1, T2, W1, W2, ND, OFF):
    NB = max(1, (T - 1).bit_length())

    def row_kernel(pk_ref, vd_ref, t1_ref, t2_ref, w2c_ref, w1_ref, b1_ref, b2_ref, out_ref):
        A = pk_ref[0]
        VD = vd_ref[0]
        dl = VD[1:2, :]
        vl = VD[0:1, :]
        iota_l = lax.broadcasted_iota(jnp.int32, (1, T), 1)
        maxd = jnp.max(dl)
        idx = jnp.min(jnp.where(dl == maxd, iota_l, T))
        cnt = jnp.sum(jnp.where((iota_l >= idx) & (vl != 0), 1, 0))

        iota1 = lax.broadcasted_iota(jnp.int32, (T1, 1), 0)
        A1 = jnp.where(iota1 < idx, A[:T1, :], 0)

        Ash = A
        for k in range(NB):
            sh = jnp.roll(Ash, -(1 << k), axis=0)
            bit = (idx >> k) & 1
            Ash = jnp.where(bit == 1, sh, Ash)
        iota2 = lax.broadcasted_iota(jnp.int32, (T2, 1), 0)
        A2 = jnp.where(iota2 < cnt, Ash[:T2, :], 0)

        iota64 = lax.broadcasted_iota(jnp.int32, (1, 64), 1)

        def embed(Ax, Tn, tab):
            ivd = Ax[:, 0:1] * ND + Ax[:, 1:2]
            parts = [jnp.where(ivd == iota64, 1.0, 0.0)]
            for k in range(3):
                parts.append(jnp.where(Ax[:, 2 + k:3 + k] == iota64, 1.0, 0.0))
            O = jnp.concatenate(parts, axis=1)
            return jnp.dot(O, tab, preferred_element_type=jnp.float32)

        x = embed(A1, T1, t1_ref[:, :])
        y = embed(A2, T2, t2_ref[:, :])

        Z = jnp.dot(y, w2c_ref[:, :], preferred_element_type=jnp.float32)
        tmod = iota2 - (iota2 // 8) * 8
        jdiv = lax.broadcasted_iota(jnp.int32, (1, 256), 1) // 32
        Zm = Z * jnp.where(jdiv == tmod, 1.0, 0.0)
        Zsel = (Zm[:, 0:32] + Zm[:, 32:64]) + (Zm[:, 64:96] + Zm[:, 96:128]) + \
            ((Zm[:, 128:160] + Zm[:, 160:192]) + (Zm[:, 192:224] + Zm[:, 224:256]))
        iota_w2 = lax.broadcasted_iota(jnp.int32, (W2, 1), 0)
        iota_t2l = lax.broadcasted_iota(jnp.int32, (1, T2), 1)
        G = jnp.where(iota_t2l // 8 == iota_w2, 1.0, 0.0)
        acc2 = jnp.dot(G, Zsel, preferred_element_type=jnp.float32) + b2_ref[:, :]

        mask2 = A1[:, 0:1] == 2
        mf = jnp.where(mask2, 1.0, 0.0)
        iota_c1 = lax.broadcasted_iota(jnp.int32, (1, T1), 1)
        Ltri = jnp.where(iota_c1 < iota1, 1.0, 0.0)
        pcum = jnp.dot(Ltri, mf, preferred_element_type=jnp.float32)
        pci = pcum.astype(jnp.int32)
        iota_w2r = lax.broadcasted_iota(jnp.int32, (1, W2), 1)
        OH = jnp.where(pci == iota_w2r, 1.0, 0.0)
        sub = jnp.dot(OH, acc2, preferred_element_type=jnp.float32)
        x = jnp.where(mask2, sub, x)

        iota_w1 = lax.broadcasted_iota(jnp.int32, (W1, 1), 0)
        iota_t1 = lax.broadcasted_iota(jnp.int32, (1, T1), 1)
        acc1 = jnp.zeros((W1, 256), jnp.float32) + b1_ref[:, :]
        for s in range(8):
            P = jnp.where(iota_t1 == iota_w1 * 8 + s, 1.0, 0.0)
            xs = jnp.dot(P, x, preferred_element_type=jnp.float32)
            acc1 = acc1 + jnp.dot(xs, w1_ref[s * 32:(s + 1) * 32, :],
                                  preferred_element_type=jnp.float32)
        out_ref[0] = acc1

    return row_kernel


def kernel(value, depth, pos, ve1, de1, se1, ve2, de2, se2, conv1_w, conv1_b, conv2_w, conv2_b):
    B, T = value.shape
    T1 = 512 + 32 * (B - 1)
    T2 = 4 * T1
    W1 = T1 // 8
    W2 = T2 // 8

    v32 = value.astype(jnp.int32)
    d32 = depth.astype(jnp.int32)
    p32 = pos.astype(jnp.int32)
    packed = jnp.concatenate([v32[:, :, None], d32[:, :, None], p32], axis=2)
    vdl = jnp.stack([v32, d32], axis=1)

    nv = ve1.shape[0]
    nd = de1.shape[0]
    ns = se1.shape[1]
    nvd = nv * nd
    OFF = (64, 128, 192)

    def pack_tab(ve, de, se):
        vd = (ve[:, None, :] + de[None, :, :]).reshape(nvd, ve.shape[1])
        vd = jnp.pad(vd, ((0, 64 - nvd), (0, 0)))
        return jnp.concatenate([vd, se[0], se[1], se[2]], axis=0)

    tab1 = pack_tab(ve1, de1, se1)
    tab2 = pack_tab(ve2, de2, se2)
    w1r = jnp.transpose(conv1_w, (2, 1, 0)).reshape(8 * conv1_w.shape[1], conv1_w.shape[0])
    w2c = jnp.transpose(conv2_w, (1, 2, 0)).reshape(conv2_w.shape[1], -1)
    b1 = conv1_b.reshape(1, -1)
    b2 = conv2_b.reshape(1, -1)

    row_kernel = _make_row_kernel(T, T1, T2, W1, W2, nd, OFF)
    out = pl.pallas_call(
        row_kernel,
        grid=(B,),
        in_specs=[
            pl.BlockSpec((1, T, 5), lambda i: (i, 0, 0)),
            pl.BlockSpec((1, 2, T), lambda i: (i, 0, 0)),
            pl.BlockSpec((256, 32), lambda i: (0, 0)),
            pl.BlockSpec((256, 32), lambda i: (0, 0)),
            pl.BlockSpec((32, 256), lambda i: (0, 0)),
            pl.BlockSpec((256, 256), lambda i: (0, 0)),
            pl.BlockSpec((1, 256), lambda i: (0, 0)),
            pl.BlockSpec((1, 32), lambda i: (0, 0)),
        ],
        out_specs=pl.BlockSpec((1, W1, 256), lambda i: (i, 0, 0)),
        out_shape=jax.ShapeDtypeStruct((B, W1, 256), jnp.float32),
        compiler_params=pltpu.CompilerParams(
            dimension_semantics=("parallel",)),
    )(packed, vdl, tab1, tab2, w2c, w1r, b1, b2)
    return out


# speedup vs baseline: 2.6738x; 1.3029x over previous
import jax
import jax.numpy as jnp
from jax import lax
from jax.experimental import pallas as pl
from jax.experimental.pallas import tpu as pltpu


def _make_row_kernel(T, T1, T2, W1, W2, ND, OFF):
    NB = max(1, (T - 1).bit_length())

    def row_kernel(pk_ref, t1_ref, t2_ref, w2c_ref, w1_ref, b1_ref, b2_ref, out_ref):
        AL = pk_ref[0]
        dl = AL[1:2, :]
        vl = AL[0:1, :]
        iota_l = lax.broadcasted_iota(jnp.int32, (1, T), 1)
        maxd = jnp.max(dl)
        idx = jnp.min(jnp.where(dl == maxd, iota_l, T))
        cnt = jnp.sum(jnp.where((iota_l >= idx) & (vl != 0), 1, 0))

        A1L = jnp.where(iota_l[:, :T1] < idx, AL[:, :T1], 0)
        A1 = jnp.transpose(A1L, (1, 0))

        Ash = AL
        for k in range(NB):
            sh = jnp.roll(Ash, -(1 << k), axis=1)
            bit = (idx >> k) & 1
            Ash = jnp.where(bit == 1, sh, Ash)
        A2L = jnp.where(iota_l[:, :T2] < cnt, Ash[:, :T2], 0)
        A2 = jnp.transpose(A2L, (1, 0))
        iota1 = lax.broadcasted_iota(jnp.int32, (T1, 1), 0)
        iota2 = lax.broadcasted_iota(jnp.int32, (T2, 1), 0)

        iota64 = lax.broadcasted_iota(jnp.int32, (1, 64), 1)

        def embed(Ax, Tn, tab):
            ivd = Ax[:, 0:1] * ND + Ax[:, 1:2]
            parts = [jnp.where(ivd == iota64, 1.0, 0.0)]
            for k in range(3):
                parts.append(jnp.where(Ax[:, 2 + k:3 + k] == iota64, 1.0, 0.0))
            O = jnp.concatenate(parts, axis=1)
            return jnp.dot(O, tab, preferred_element_type=jnp.float32)

        x = embed(A1, T1, t1_ref[:, :])
        y = embed(A2, T2, t2_ref[:, :])

        Z = jnp.dot(y, w2c_ref[:, :], preferred_element_type=jnp.float32)
        tmod = iota2 - (iota2 // 8) * 8
        jdiv = lax.broadcasted_iota(jnp.int32, (1, 256), 1) // 32
        Zm = Z * jnp.where(jdiv == tmod, 1.0, 0.0)
        Zsel = (Zm[:, 0:32] + Zm[:, 32:64]) + (Zm[:, 64:96] + Zm[:, 96:128]) + \
            ((Zm[:, 128:160] + Zm[:, 160:192]) + (Zm[:, 192:224] + Zm[:, 224:256]))
        iota_w2 = lax.broadcasted_iota(jnp.int32, (W2, 1), 0)
        iota_t2l = lax.broadcasted_iota(jnp.int32, (1, T2), 1)
        G = jnp.where(iota_t2l // 8 == iota_w2, 1.0, 0.0)
        acc2 = jnp.dot(G, Zsel, preferred_element_type=jnp.float32) + b2_ref[:, :]

        mask2 = A1[:, 0:1] == 2
        mf = jnp.where(mask2, 1.0, 0.0)
        iota_c1 = lax.broadcasted_iota(jnp.int32, (1, T1), 1)
        Ltri = jnp.where(iota_c1 < iota1, 1.0, 0.0)
        pcum = jnp.dot(Ltri, mf, preferred_element_type=jnp.float32)
        pci = pcum.astype(jnp.int32)
        iota_w2r = lax.broadcasted_iota(jnp.int32, (1, W2), 1)
        OH = jnp.where(pci == iota_w2r, 1.0, 0.0)
        sub = jnp.dot(OH, acc2, preferred_element_type=jnp.float32)
        x = jnp.where(mask2, sub, x)

        iota_w1 = lax.broadcasted_iota(jnp.int32, (W1, 1), 0)
        iota_t1 = lax.broadcasted_iota(jnp.int32, (1, T1), 1)
        acc1 = jnp.zeros((W1, 256), jnp.float32) + b1_ref[:, :]
        for s in range(8):
            P = jnp.where(iota_t1 == iota_w1 * 8 + s, 1.0, 0.0)
            xs = jnp.dot(P, x, preferred_element_type=jnp.float32)
            acc1 = acc1 + jnp.dot(xs, w1_ref[s * 32:(s + 1) * 32, :],
                                  preferred_element_type=jnp.float32)
        out_ref[0] = acc1

    return row_kernel


def kernel(value, depth, pos, ve1, de1, se1, ve2, de2, se2, conv1_w, conv1_b, conv2_w, conv2_b):
    B, T = value.shape
    T1 = 512 + 32 * (B - 1)
    T2 = 4 * T1
    W1 = T1 // 8
    W2 = T2 // 8

    v32 = value.astype(jnp.int32)
    d32 = depth.astype(jnp.int32)
    p32 = pos.astype(jnp.int32)
    packed = jnp.concatenate(
        [v32[:, None, :], d32[:, None, :], jnp.transpose(p32, (0, 2, 1))],
        axis=1)

    nv = ve1.shape[0]
    nd = de1.shape[0]
    ns = se1.shape[1]
    nvd = nv * nd
    OFF = (64, 128, 192)

    def pack_tab(ve, de, se):
        vd = (ve[:, None, :] + de[None, :, :]).reshape(nvd, ve.shape[1])
        vd = jnp.pad(vd, ((0, 64 - nvd), (0, 0)))
        return jnp.concatenate([vd, se[0], se[1], se[2]], axis=0)

    tab1 = pack_tab(ve1, de1, se1)
    tab2 = pack_tab(ve2, de2, se2)
    w1r = jnp.transpose(conv1_w, (2, 1, 0)).reshape(8 * conv1_w.shape[1], conv1_w.shape[0])
    w2c = jnp.transpose(conv2_w, (1, 2, 0)).reshape(conv2_w.shape[1], -1)
    b1 = conv1_b.reshape(1, -1)
    b2 = conv2_b.reshape(1, -1)

    row_kernel = _make_row_kernel(T, T1, T2, W1, W2, nd, OFF)
    out = pl.pallas_call(
        row_kernel,
        grid=(B,),
        in_specs=[
            pl.BlockSpec((1, 5, T), lambda i: (i, 0, 0)),
            pl.BlockSpec((256, 32), lambda i: (0, 0)),
            pl.BlockSpec((256, 32), lambda i: (0, 0)),
            pl.BlockSpec((32, 256), lambda i: (0, 0)),
            pl.BlockSpec((256, 256), lambda i: (0, 0)),
            pl.BlockSpec((1, 256), lambda i: (0, 0)),
            pl.BlockSpec((1, 32), lambda i: (0, 0)),
        ],
        out_specs=pl.BlockSpec((1, W1, 256), lambda i: (i, 0, 0)),
        out_shape=jax.ShapeDtypeStruct((B, W1, 256), jnp.float32),
        compiler_params=pltpu.CompilerParams(
            dimension_semantics=("parallel",)),
    )(packed, tab1, tab2, w2c, w1r, b1, b2)
    return out
